# baseline (speedup 1.0000x reference)
import functools

import jax
import jax.numpy as jnp
from jax import lax
from jax.experimental import pallas as pl
from jax.experimental.pallas import tpu as pltpu
from jax.experimental.pallas import tpu_sc as plsc

B = 16
L = 2048
D = 1024
N = 16384

NC = 2
NS = 16
NW = NC * NS
RPW = (B * L) // NW
G = 32
NCHUNK = RPW // G


def _sc_body(src_hbm, batch_hbm, zeros_hbm, out_hbm, mask_hbm,
             batch_v, bufA, bufB, zbuf, maskbuf):
    wid = lax.axis_index("s") * NC + lax.axis_index("c")
    b = wid // 2
    p0 = (wid % 2) * RPW

    pltpu.sync_copy(batch_hbm, batch_v)
    pltpu.sync_copy(zeros_hbm, zbuf)

    z16 = jnp.zeros((16,), jnp.int32)
    one16 = jnp.full((16,), 1, jnp.int32)
    bvec = jnp.full((16,), b, jnp.int32)

    def scan_body(i, carry):
        lt, le = carry
        v = batch_v[pl.ds(i * 16, 16)]
        lt = lt + jnp.where(v < bvec, one16, z16)
        le = le + jnp.where(v <= bvec, one16, z16)
        return lt, le

    lt, le = lax.fori_loop(0, N // 16, scan_body, (z16, z16))
    start_b = jnp.sum(lt)
    end_b = jnp.sum(le)
    count_b = end_b - start_b

    valid = jnp.clip(count_b - p0, 0, RPW)

    iota16 = lax.iota(jnp.int32, 16)
    cb = jnp.full((16,), count_b, jnp.int32)

    def mask_body(j, _):
        p = jnp.full((16,), p0 + j * 16, jnp.int32) + iota16
        maskbuf[pl.ds(j * 16, 16)] = jnp.where(p < cb, one16, z16)
        return 0

    lax.fori_loop(0, RPW // 16, mask_body, 0)
    pltpu.sync_copy(maskbuf, mask_hbm.at[pl.ds(wid * RPW, RPW)])

    src0 = start_b + p0

    def chunk_body(j, _):
        r0 = j * G
        o0 = wid * RPW + r0
        nvalid = valid - r0

        @pl.when(nvalid >= G)
        def _full():
            pltpu.sync_copy(src_hbm.at[pl.ds(src0 + r0, G)], bufA)
            pltpu.sync_copy(bufA, out_hbm.at[pl.ds(o0, G)])

        @pl.when(nvalid <= 0)
        def _empty():
            pltpu.sync_copy(zbuf, out_hbm.at[pl.ds(o0, G)])

        @pl.when((nvalid > 0) & (nvalid < G))
        def _partial():
            pltpu.sync_copy(zbuf, out_hbm.at[pl.ds(o0, G)])

            def row_body(r, _):
                @pl.when(r < nvalid)
                def _():
                    pltpu.sync_copy(src_hbm.at[pl.ds(src0 + r0 + r, 1)],
                                    bufB.at[pl.ds(0, 1)])
                    pltpu.sync_copy(bufB.at[pl.ds(0, 1)],
                                    out_hbm.at[pl.ds(o0 + r, 1)])
                return 0

            lax.fori_loop(0, G, row_body, 0)

        return 0

    lax.fori_loop(0, NCHUNK, chunk_body, 0)


@functools.partial(
    pl.kernel,
    out_type=(jax.ShapeDtypeStruct((B * L, D), jnp.float32),
              jax.ShapeDtypeStruct((B * L,), jnp.int32)),
    mesh=plsc.VectorSubcoreMesh(core_axis_name="c", subcore_axis_name="s",
                                num_cores=NC, num_subcores=NS),
    scratch_types=[
        pltpu.VMEM((N,), jnp.int32),
        pltpu.VMEM((G, D), jnp.float32),
        pltpu.VMEM((G, D), jnp.float32),
        pltpu.VMEM((G, D), jnp.float32),
        pltpu.VMEM((RPW,), jnp.int32),
    ],
    compiler_params=pltpu.CompilerParams(use_tc_tiling_on_sc=False,
                                         needs_layout_passes=False),
)
def _sc_kernel(src_hbm, batch_hbm, zeros_hbm, out_hbm, mask_hbm,
               batch_v, bufA, bufB, zbuf, maskbuf):
    _sc_body(src_hbm, batch_hbm, zeros_hbm, out_hbm, mask_hbm,
             batch_v, bufA, bufB, zbuf, maskbuf)


@jax.jit
def kernel(src, batch):
    zeros = jnp.zeros((G, D), jnp.float32)
    padded_flat, mask_flat = _sc_kernel(src, batch.astype(jnp.int32), zeros)
    return padded_flat.reshape(B, L, D), mask_flat.reshape(B, L) != 0
